# Initial kernel scaffold; baseline (speedup 1.0000x reference)
import jax
import jax.numpy as jnp
from jax.experimental import pallas as pl


def kernel(x, Pi, centroids):
    raise NotImplementedError("write your pallas kernel here")


# speedup vs baseline: 5.7690x; 5.7690x over previous
import jax
import jax.numpy as jnp
from jax.experimental import pallas as pl

_DIM = 2048
_BT = 512


def _fused_kernel(x_ref, pi_ref, c_ref, out_ref):
    x = x_ref[...]
    norm = jnp.sqrt(jnp.sum(x * x, axis=1, keepdims=True))
    unit = x * (1.0 / (norm + 1e-10))
    y = jnp.dot(unit, pi_ref[...], preferred_element_type=jnp.float32)

    c = [c_ref[0, i] for i in range(8)]
    y_hat = jnp.full_like(y, c[0])
    for i in range(7):
        b = (c[i] + c[i + 1]) * 0.5
        y_hat += jnp.where(y > b, c[i + 1] - c[i], 0.0)

    x_hat = jax.lax.dot_general(
        y_hat, pi_ref[...],
        dimension_numbers=(((1,), (1,)), ((), ())),
        preferred_element_type=jnp.float32,
    )
    out_ref[...] = x_hat * norm


def kernel(x, Pi, centroids):
    n_tok, dim = x.shape
    c2d = centroids.reshape(1, -1)
    grid = (n_tok // _BT,)
    return pl.pallas_call(
        _fused_kernel,
        grid=grid,
        in_specs=[
            pl.BlockSpec((_BT, dim), lambda i: (i, 0)),
            pl.BlockSpec((dim, dim), lambda i: (0, 0)),
            pl.BlockSpec((1, 8), lambda i: (0, 0)),
        ],
        out_specs=pl.BlockSpec((_BT, dim), lambda i: (i, 0)),
        out_shape=jax.ShapeDtypeStruct((n_tok, dim), jnp.float32),
    )(x, Pi, c2d)
